# baseline (speedup 1.0000x reference)
import functools

import jax
import jax.numpy as jnp
from jax import lax
from jax.experimental import pallas as pl
from jax.experimental.pallas import tpu as pltpu
from jax.experimental.pallas import tpu_sc as plsc

NC = 2
NS = 16
NW = NC * NS
L = 16

B, N, D = 4, 8192, 1024
PER_W = N // NW
CHUNK_ROWS = 32
CHUNK = CHUNK_ROWS * D
NCHUNKS = PER_W // CHUNK_ROWS


def _sc_body(x_hbm, pos_hbm, out_hbm, xbuf, pbuf):
    c = lax.axis_index("c")
    s = lax.axis_index("s")
    wid = s * NC + c
    pbase = wid * (PER_W * D)

    def chunk_body(ci, carry):
        poff = pbase + ci * CHUNK
        pltpu.sync_copy(pos_hbm.at[pl.ds(poff, CHUNK)], pbuf)
        for b in range(B):
            xoff = b * (N * D) + poff
            pltpu.sync_copy(x_hbm.at[pl.ds(xoff, CHUNK)], xbuf)

            def add_body(i, acc):
                sl = pl.ds(pl.multiple_of(i * L, L), L)
                xbuf[sl] = xbuf[sl] + pbuf[sl]
                return acc

            lax.fori_loop(0, CHUNK // L, add_body, 0)
            pltpu.sync_copy(xbuf, out_hbm.at[pl.ds(xoff, CHUNK)])
        return carry

    lax.fori_loop(0, NCHUNKS, chunk_body, 0)


_sc_call = functools.partial(
    pl.kernel,
    out_type=jax.ShapeDtypeStruct((B * N * D,), jnp.float32),
    mesh=plsc.VectorSubcoreMesh(
        core_axis_name="c", subcore_axis_name="s",
        num_cores=NC, num_subcores=NS),
    scratch_types=[
        pltpu.VMEM((CHUNK,), jnp.float32),
        pltpu.VMEM((CHUNK,), jnp.float32),
    ],
)(_sc_body)


def kernel(x, pos_emb):
    b, n, d = x.shape
    out_flat = _sc_call(x.reshape(-1), pos_emb[:n].reshape(-1))
    return out_flat.reshape(b, n, d)


# speedup vs baseline: 1.6840x; 1.6840x over previous
import functools

import jax
import jax.numpy as jnp
from jax import lax
from jax.experimental import pallas as pl
from jax.experimental.pallas import tpu as pltpu
from jax.experimental.pallas import tpu_sc as plsc

NC = 2
NS = 16
NW = NC * NS
L = 16

B, N, D = 4, 8192, 1024
PER_W = N // NW
CHUNK_ROWS = 16
CHUNK = CHUNK_ROWS * D
NCH = PER_W // CHUNK_ROWS


def _sc_body(x_hbm, pos_hbm, out_hbm,
             xb0, xb1, xb2, xb3, pb0, pb1,
             sx0, sx1, sx2, sx3, so0, so1, so2, so3, sp0, sp1):
    c = lax.axis_index("c")
    s = lax.axis_index("s")
    wid = s * NC + c
    pbase = wid * (PER_W * D)

    xbs = (xb0, xb1, xb2, xb3)
    sxs = (sx0, sx1, sx2, sx3)
    sos = (so0, so1, so2, so3)
    pbs = (pb0, pb1)
    sps = (sp0, sp1)

    def poff(ci):
        return pbase + ci * CHUNK

    def xoff(ci, b):
        return b * (N * D) + poff(ci)

    pltpu.async_copy(pos_hbm.at[pl.ds(poff(0), CHUNK)], pb0, sp0)
    for b in range(B):
        pltpu.async_copy(x_hbm.at[pl.ds(xoff(0, b), CHUNK)], xbs[b], sxs[b])

    def pair_body(cp, carry):
        for cc in range(2):
            ci = 2 * cp + cc
            if cc == 0:
                pltpu.async_copy(
                    pos_hbm.at[pl.ds(poff(ci + 1), CHUNK)], pbs[1], sps[1])
            else:
                @pl.when(cp + 1 < NCH // 2)
                def _():
                    pltpu.async_copy(
                        pos_hbm.at[pl.ds(poff(ci + 1), CHUNK)], pbs[0], sps[0])

            @pl.when(ci > 0)
            def _():
                for b in range(B):
                    pltpu.make_async_copy(
                        xbs[b], out_hbm.at[pl.ds(0, CHUNK)], sos[b]).wait()
                    pltpu.async_copy(
                        x_hbm.at[pl.ds(xoff(ci, b), CHUNK)], xbs[b], sxs[b])

            pltpu.make_async_copy(
                pos_hbm.at[pl.ds(0, CHUNK)], pbs[cc], sps[cc]).wait()

            for b in range(B):
                pltpu.make_async_copy(
                    x_hbm.at[pl.ds(0, CHUNK)], xbs[b], sxs[b]).wait()

                xb, pb = xbs[b], pbs[cc]

                @plsc.parallel_loop(0, CHUNK, step=L, unroll=8)
                def _(i):
                    sl = pl.ds(pl.multiple_of(i, L), L)
                    xb[sl] = xb[sl] + pb[sl]

                pltpu.async_copy(
                    xb, out_hbm.at[pl.ds(xoff(ci, b), CHUNK)], sos[b])
        return carry

    lax.fori_loop(0, NCH // 2, pair_body, 0)

    for b in range(B):
        pltpu.make_async_copy(
            xbs[b], out_hbm.at[pl.ds(0, CHUNK)], sos[b]).wait()


_sc_call = functools.partial(
    pl.kernel,
    out_type=jax.ShapeDtypeStruct((B * N * D,), jnp.float32),
    mesh=plsc.VectorSubcoreMesh(
        core_axis_name="c", subcore_axis_name="s",
        num_cores=NC, num_subcores=NS),
    scratch_types=(
        [pltpu.VMEM((CHUNK,), jnp.float32)] * 6
        + [pltpu.SemaphoreType.DMA] * 10
    ),
)(_sc_body)


def kernel(x, pos_emb):
    b, n, d = x.shape
    out_flat = _sc_call(x.reshape(-1), pos_emb[:n].reshape(-1))
    return out_flat.reshape(b, n, d)
